# Initial kernel scaffold; baseline (speedup 1.0000x reference)
import jax
import jax.numpy as jnp
from jax.experimental import pallas as pl


def kernel(w_Ak, b_Ak, w_tildeAk, b_tildeAk, w_Wk, b_Wk, w_Lk, b_Lk, adaptiveScalar_in, adaptiveScalar_out, x_0, x_kk, x_k):
    raise NotImplementedError("write your pallas kernel here")


# speedup vs baseline: 31.9267x; 31.9267x over previous
import functools

import jax
import jax.numpy as jnp
from jax.experimental import pallas as pl
from jax.experimental.pallas import tpu as pltpu

NEG_SLOPE = 0.15
BN_EPS = 1e-5


def _leaky(x):
    return jnp.where(x >= 0, x, NEG_SLOPE * x)


def _w_mat(w):
    return jnp.transpose(w, (2, 3, 1, 0)).reshape(-1, w.shape[0])


def _phases(x_nchw):
    n, c, h, w = x_nchw.shape
    xp = jnp.pad(x_nchw, ((0, 0), (0, 0), (1, 1), (1, 1)))
    xr = xp.reshape(n, c, h // 2 + 1, 2, w // 2 + 1, 2)
    ph = jnp.transpose(xr, (0, 3, 5, 2, 4, 1)).astype(jnp.bfloat16)
    return ph.reshape(n * 4, h // 2 + 1, w // 2 + 1, c)


def _stage1_kernel(c_up, ho, wo,
                   p0_ref, wta_ref, bta_ref,
                   pkk_ref, wa_ref, ba_ref,
                   pk_ref, wlw_ref, blw_ref,
                   sout_ref, out_ref, stats_ref):
    m = ho * wo

    def patches(pref):
        p = pref[...]
        cols = []
        for dh in range(3):
            for dw in range(3):
                sl = p[2 * (dh & 1) + (dw & 1),
                       (dh >> 1):(dh >> 1) + ho,
                       (dw >> 1):(dw >> 1) + wo, :]
                cols.append(sl.reshape(m, sl.shape[-1]))
        return jnp.concatenate(cols, axis=1)

    tA = jnp.dot(patches(p0_ref), wta_ref[...],
                 preferred_element_type=jnp.float32) + bta_ref[...]
    A = jnp.dot(patches(pkk_ref), wa_ref[...],
                preferred_element_type=jnp.float32) + ba_ref[...]
    LW = jnp.dot(patches(pk_ref), wlw_ref[...],
                 preferred_element_type=jnp.float32) + blw_ref[...]

    Lk = LW[:, :c_up]
    Wk = LW[:, c_up:]
    low = jnp.exp(tA) * sout_ref[...] + jnp.maximum(_leaky(A), Wk)

    out_ref[:, :c_up] = Lk.astype(jnp.bfloat16)
    out_ref[:, c_up:] = low.astype(jnp.bfloat16)

    stats_ref[0, 0:1, :c_up] = jnp.sum(Lk, axis=0, keepdims=True)
    stats_ref[0, 1:2, :c_up] = jnp.sum(Lk * Lk, axis=0, keepdims=True)
    stats_ref[0, 0:1, c_up:] = jnp.sum(low, axis=0, keepdims=True)
    stats_ref[0, 1:2, c_up:] = jnp.sum(low * low, axis=0, keepdims=True)


def _stage2_kernel(x_ref, st_ref, o_ref):
    y = (x_ref[...].astype(jnp.float32) - st_ref[0:1, :]) * st_ref[1:2, :]
    y = jnp.where(y >= 0, y, NEG_SLOPE * y)
    o_ref[0] = jnp.transpose(y, (1, 0))


@jax.jit
def _mcn_forward(x_0, x_kk, x_k, params):
    n, c0, h, w = x_0.shape
    ho, wo = h // 2, w // 2
    m_img = ho * wo

    p0 = _phases(x_0)
    pkk = _phases(x_kk)
    pk = _phases(x_k)

    c_up = params["w_Lk"].shape[0]
    c_low = params["w_Wk"].shape[0]
    c_tot = c_up + c_low

    s_in = params["adaptiveScalar_in"].astype(jnp.float32)
    s_out = params["adaptiveScalar_out"].astype(jnp.float32)

    w_tA = (_w_mat(params["w_tildeAk"]) * s_in).astype(jnp.bfloat16)
    b_tA = (params["b_tildeAk"] * s_in)[None, :]
    w_A = _w_mat(params["w_Ak"]).astype(jnp.bfloat16)
    b_A = params["b_Ak"][None, :]
    w_LW = jnp.concatenate([_w_mat(params["w_Lk"]),
                            _w_mat(params["w_Wk"])], axis=1).astype(jnp.bfloat16)
    b_LW = jnp.concatenate([params["b_Lk"], params["b_Wk"]])[None, :]
    s_out_vec = jnp.broadcast_to(s_out, (1, c_low)).astype(jnp.float32)

    k0 = p0.shape[-1]
    kkk = pkk.shape[-1]
    kk = pk.shape[-1]

    def phase_spec(c):
        return pl.BlockSpec((4, ho + 1, wo + 1, c), lambda i: (i, 0, 0, 0))

    def full_spec(r, c):
        return pl.BlockSpec((r, c), lambda i: (0, 0))

    cparams = pltpu.CompilerParams(
        dimension_semantics=("parallel",),
        vmem_limit_bytes=64 * 1024 * 1024)

    kernel1 = functools.partial(_stage1_kernel, c_up, ho, wo)

    pre_bn, partial_stats = pl.pallas_call(
        kernel1,
        grid=(n,),
        out_shape=(jax.ShapeDtypeStruct((n * m_img, c_tot), jnp.bfloat16),
                   jax.ShapeDtypeStruct((n, 2, c_tot), jnp.float32)),
        in_specs=[phase_spec(k0), full_spec(9 * k0, c_low), full_spec(1, c_low),
                  phase_spec(kkk), full_spec(9 * kkk, c_low), full_spec(1, c_low),
                  phase_spec(kk), full_spec(9 * kk, c_tot), full_spec(1, c_tot),
                  full_spec(1, c_low)],
        out_specs=(pl.BlockSpec((m_img, c_tot), lambda i: (i, 0)),
                   pl.BlockSpec((1, 2, c_tot), lambda i: (i, 0, 0))),
        compiler_params=cparams,
    )(p0, w_tA, b_tA, pkk, w_A, b_A, pk, w_LW, b_LW, s_out_vec)

    m = n * m_img
    sums = jnp.sum(partial_stats[:, 0, :], axis=0)
    sqs = jnp.sum(partial_stats[:, 1, :], axis=0)
    mean = sums / m
    var = jnp.maximum(sqs / m - mean * mean, 0.0)
    inv = jax.lax.rsqrt(var + BN_EPS)
    st = jnp.stack([mean, inv], axis=0)

    cb = 128 if c_tot % 128 == 0 else c_tot
    out = pl.pallas_call(
        _stage2_kernel,
        grid=(n, c_tot // cb),
        out_shape=jax.ShapeDtypeStruct((n, c_tot, m_img), jnp.float32),
        in_specs=[pl.BlockSpec((m_img, cb), lambda i, j: (i, j)),
                  pl.BlockSpec((2, cb), lambda i, j: (0, j))],
        out_specs=pl.BlockSpec((1, cb, m_img), lambda i, j: (i, j, 0)),
        compiler_params=pltpu.CompilerParams(
            dimension_semantics=("parallel", "parallel"),
            vmem_limit_bytes=64 * 1024 * 1024),
    )(pre_bn, st)

    return out.reshape(n, c_tot, ho, wo)


def kernel(w_Ak, b_Ak, w_tildeAk, b_tildeAk, w_Wk, b_Wk, w_Lk, b_Lk,
           adaptiveScalar_in, adaptiveScalar_out, x_0, x_kk, x_k):
    params = {
        "w_Ak": w_Ak, "b_Ak": b_Ak,
        "w_tildeAk": w_tildeAk, "b_tildeAk": b_tildeAk,
        "w_Wk": w_Wk, "b_Wk": b_Wk,
        "w_Lk": w_Lk, "b_Lk": b_Lk,
        "adaptiveScalar_in": adaptiveScalar_in,
        "adaptiveScalar_out": adaptiveScalar_out,
    }
    return _mcn_forward(x_0, x_kk, x_k, params)


# speedup vs baseline: 43.2703x; 1.3553x over previous
import functools

import jax
import jax.numpy as jnp
from jax.experimental import pallas as pl
from jax.experimental.pallas import tpu as pltpu

NEG_SLOPE = 0.15
BN_EPS = 1e-5


def _leaky(x):
    return jnp.where(x >= 0, x, NEG_SLOPE * x)


def _w_mat(w):
    return jnp.transpose(w, (2, 3, 1, 0)).reshape(-1, w.shape[0])


def _phases(x_nchw):
    n, c, h, w = x_nchw.shape
    xr = x_nchw.reshape(n, c, h // 2, 2, w // 2, 2)
    ph = jnp.transpose(xr, (0, 3, 5, 2, 4, 1)).astype(jnp.bfloat16)
    return ph.reshape(n * 4, h // 2, w // 2, c)


def _stage1_kernel(c_up, ho, wo,
                   p0_ref, wta_ref, bta_ref,
                   pkk_ref, wa_ref, ba_ref,
                   pk_ref, wlw_ref, blw_ref,
                   sout_ref, out_ref, stats_ref):
    m = ho * wo

    def patches(pref):
        p = pref[...]
        c = p.shape[-1]
        cols = []
        for dh in range(3):
            for dw in range(3):
                ph = 0 if dh == 1 else 1
                pw = 0 if dw == 1 else 1
                sl = p[2 * ph + pw]
                if dh == 0:
                    sl = jnp.concatenate(
                        [jnp.zeros((1, wo, c), sl.dtype), sl[:-1]], axis=0)
                if dw == 0:
                    sl = jnp.concatenate(
                        [jnp.zeros((ho, 1, c), sl.dtype), sl[:, :-1]], axis=1)
                cols.append(sl.reshape(m, c))
        return jnp.concatenate(cols, axis=1)

    tA = jnp.dot(patches(p0_ref), wta_ref[...],
                 preferred_element_type=jnp.float32) + bta_ref[...]
    A = jnp.dot(patches(pkk_ref), wa_ref[...],
                preferred_element_type=jnp.float32) + ba_ref[...]
    LW = jnp.dot(patches(pk_ref), wlw_ref[...],
                 preferred_element_type=jnp.float32) + blw_ref[...]

    Lk = LW[:, :c_up]
    Wk = LW[:, c_up:]
    low = jnp.exp(tA) * sout_ref[...] + jnp.maximum(_leaky(A), Wk)

    out_ref[:, :c_up] = Lk.astype(jnp.bfloat16)
    out_ref[:, c_up:] = low.astype(jnp.bfloat16)

    stats_ref[0, 0:1, :c_up] = jnp.sum(Lk, axis=0, keepdims=True)
    stats_ref[0, 1:2, :c_up] = jnp.sum(Lk * Lk, axis=0, keepdims=True)
    stats_ref[0, 0:1, c_up:] = jnp.sum(low, axis=0, keepdims=True)
    stats_ref[0, 1:2, c_up:] = jnp.sum(low * low, axis=0, keepdims=True)


def _stage2_kernel(x_ref, st_ref, o_ref):
    y = (x_ref[...].astype(jnp.float32) - st_ref[0:1, :]) * st_ref[1:2, :]
    y = jnp.where(y >= 0, y, NEG_SLOPE * y)
    o_ref[0] = jnp.transpose(y, (1, 0))


@jax.jit
def _mcn_forward(x_0, x_kk, x_k, params):
    n, c0, h, w = x_0.shape
    ho, wo = h // 2, w // 2
    m_img = ho * wo

    p0 = _phases(x_0)
    pkk = _phases(x_kk)
    pk = _phases(x_k)

    c_up = params["w_Lk"].shape[0]
    c_low = params["w_Wk"].shape[0]
    c_tot = c_up + c_low

    s_in = params["adaptiveScalar_in"].astype(jnp.float32)
    s_out = params["adaptiveScalar_out"].astype(jnp.float32)

    w_tA = (_w_mat(params["w_tildeAk"]) * s_in).astype(jnp.bfloat16)
    b_tA = (params["b_tildeAk"] * s_in)[None, :]
    w_A = _w_mat(params["w_Ak"]).astype(jnp.bfloat16)
    b_A = params["b_Ak"][None, :]
    w_LW = jnp.concatenate([_w_mat(params["w_Lk"]),
                            _w_mat(params["w_Wk"])], axis=1).astype(jnp.bfloat16)
    b_LW = jnp.concatenate([params["b_Lk"], params["b_Wk"]])[None, :]
    s_out_vec = jnp.broadcast_to(s_out, (1, c_low)).astype(jnp.float32)

    k0 = p0.shape[-1]
    kkk = pkk.shape[-1]
    kk = pk.shape[-1]

    def phase_spec(c):
        return pl.BlockSpec((4, ho, wo, c), lambda i: (i, 0, 0, 0))

    def full_spec(r, c):
        return pl.BlockSpec((r, c), lambda i: (0, 0))

    cparams = pltpu.CompilerParams(
        dimension_semantics=("parallel",),
        vmem_limit_bytes=64 * 1024 * 1024)

    kernel1 = functools.partial(_stage1_kernel, c_up, ho, wo)

    pre_bn, partial_stats = pl.pallas_call(
        kernel1,
        grid=(n,),
        out_shape=(jax.ShapeDtypeStruct((n * m_img, c_tot), jnp.bfloat16),
                   jax.ShapeDtypeStruct((n, 2, c_tot), jnp.float32)),
        in_specs=[phase_spec(k0), full_spec(9 * k0, c_low), full_spec(1, c_low),
                  phase_spec(kkk), full_spec(9 * kkk, c_low), full_spec(1, c_low),
                  phase_spec(kk), full_spec(9 * kk, c_tot), full_spec(1, c_tot),
                  full_spec(1, c_low)],
        out_specs=(pl.BlockSpec((m_img, c_tot), lambda i: (i, 0)),
                   pl.BlockSpec((1, 2, c_tot), lambda i: (i, 0, 0))),
        compiler_params=cparams,
    )(p0, w_tA, b_tA, pkk, w_A, b_A, pk, w_LW, b_LW, s_out_vec)

    m = n * m_img
    sums = jnp.sum(partial_stats[:, 0, :], axis=0)
    sqs = jnp.sum(partial_stats[:, 1, :], axis=0)
    mean = sums / m
    var = jnp.maximum(sqs / m - mean * mean, 0.0)
    inv = jax.lax.rsqrt(var + BN_EPS)
    st = jnp.stack([mean, inv], axis=0)

    cb = 128 if c_tot % 128 == 0 else c_tot
    out = pl.pallas_call(
        _stage2_kernel,
        grid=(n, c_tot // cb),
        out_shape=jax.ShapeDtypeStruct((n, c_tot, m_img), jnp.float32),
        in_specs=[pl.BlockSpec((m_img, cb), lambda i, j: (i, j)),
                  pl.BlockSpec((2, cb), lambda i, j: (0, j))],
        out_specs=pl.BlockSpec((1, cb, m_img), lambda i, j: (i, j, 0)),
        compiler_params=pltpu.CompilerParams(
            dimension_semantics=("parallel", "parallel"),
            vmem_limit_bytes=64 * 1024 * 1024),
    )(pre_bn, st)

    return out.reshape(n, c_tot, ho, wo)


def kernel(w_Ak, b_Ak, w_tildeAk, b_tildeAk, w_Wk, b_Wk, w_Lk, b_Lk,
           adaptiveScalar_in, adaptiveScalar_out, x_0, x_kk, x_k):
    params = {
        "w_Ak": w_Ak, "b_Ak": b_Ak,
        "w_tildeAk": w_tildeAk, "b_tildeAk": b_tildeAk,
        "w_Wk": w_Wk, "b_Wk": b_Wk,
        "w_Lk": w_Lk, "b_Lk": b_Lk,
        "adaptiveScalar_in": adaptiveScalar_in,
        "adaptiveScalar_out": adaptiveScalar_out,
    }
    return _mcn_forward(x_0, x_kk, x_k, params)
